# baseline (speedup 1.0000x reference)
import jax
import jax.numpy as jnp
from jax import lax
from jax.experimental import pallas as pl
from jax.experimental.pallas import tpu as pltpu
from jax.experimental.pallas import tpu_sc as plsc

N_NODES = 1000
N_PAD = 1024
NPS = 64
LAST = N_NODES - 15 * NPS
E_PAD = 112
EG = E_PAD // 16
IN_D = 128
OUT_D = 16
L = 16


def _rsqrt(q):
    y = plsc.bitcast(
        jnp.int32(0x5F3759DF) - (plsc.bitcast(q, jnp.int32) >> 1),
        jnp.float32)
    for _ in range(3):
        y = y * (1.5 - 0.5 * q * y * y)
    return y


def _sc_body(x_hbm, wtp_hbm, bias_hbm, src_hbm, dst_hbm,
             out_hbm,
             x_v, wtp_v, bias_v, src_v, dst_v,
             degall_v, xrows_v, o8_v, sem_g):
    s = lax.axis_index("s")
    base = s * NPS
    xoff = lax.min(base, jnp.int32(N_NODES - NPS))
    xshift = base - xoff

    pltpu.sync_copy(src_hbm, src_v)
    pltpu.sync_copy(dst_hbm, dst_v)

    gathers = [
        pltpu.async_copy(x_hbm.at[src_v.at[g]],
                         xrows_v.at[pl.ds(g * L, L), :], sem_g)
        for g in range(EG)
    ]

    pltpu.sync_copy(x_hbm.at[pl.ds(xoff, NPS), :], x_v)
    pltpu.sync_copy(wtp_hbm, wtp_v)
    pltpu.sync_copy(bias_hbm, bias_v)

    ones = jnp.full((L,), 1.0, jnp.float32)
    for i in range(N_PAD // L):
        degall_v[pl.ds(i * L, L)] = ones
    for g in range(EG):
        sv = src_v[g, :]
        dv = dst_v[g, :]
        keep = sv != dv
        plsc.addupdate_scatter(degall_v, [dv], ones, mask=keep)

    bias_vec = bias_v[...]

    def mm_body(t, carry):
        m0 = t * L
        dsq = 1.0 / degall_v[pl.ds(base + m0, L)]
        for sub in range(4):
            n0 = m0 + sub * 4
            accs = [[jnp.zeros((L,), jnp.float32) for _ in range(4)]
                    for _ in range(4)]
            for kb in range(IN_D // L):
                xvs = [x_v[n0 + u + xshift, pl.ds(kb * L, L)]
                       for u in range(4)]
                for j in range(L):
                    k = kb * L + j
                    wrow = wtp_v[k >> 3, pl.ds((k & 7) * L, L)]
                    for u in range(4):
                        b = j % 4
                        accs[u][b] = accs[u][b] + xvs[u][j] * wrow
            for u in range(4):
                n = n0 + u
                a = accs[u]
                acc = (a[0] + a[1]) + (a[2] + a[3])
                o8_v[n >> 3, pl.ds((n & 7) * L, L)] = (
                    acc * dsq[sub * 4 + u] + bias_vec)
        return carry

    lax.fori_loop(0, NPS // L, mm_body, 0)

    for g_desc in gathers:
        g_desc.wait()

    def edge_body(g, carry):
        sv = src_v[g, :]
        dv = dst_v[g, :]
        deg_s = plsc.load_gather(degall_v, [sv])
        deg_d = plsc.load_gather(degall_v, [dv])
        nv = _rsqrt(deg_s * deg_d)
        ldv = dv - base
        for j in range(L):
            lj = ldv[j]
            sj = sv[j]
            dj = dv[j]
            nj = nv[j]

            @pl.when((lj >= 0) & (lj < NPS) & (sj != dj))
            def _():
                erow = g * L + j

                def emm(kb, acc4):
                    xv = xrows_v[erow, pl.ds(kb * L, L)]
                    acc4 = list(acc4)
                    for jj in range(L):
                        k8 = jj
                        wrow = wtp_v[(kb * L + jj) >> 3,
                                     pl.ds(((kb * L + jj) & 7) * L, L)]
                        acc4[jj % 4] = acc4[jj % 4] + xv[jj] * wrow
                    return tuple(acc4)

                a = lax.fori_loop(0, IN_D // L, emm,
                                  tuple(jnp.zeros((L,), jnp.float32)
                                        for _ in range(4)))
                msg = ((a[0] + a[1]) + (a[2] + a[3])) * nj
                cur = o8_v[lj >> 3, pl.ds((lj & 7) * L, L)]
                o8_v[lj >> 3, pl.ds((lj & 7) * L, L)] = cur + msg

        return carry

    lax.fori_loop(0, EG, edge_body, 0)

    @pl.when(s < 15)
    def _():
        pltpu.sync_copy(o8_v, out_hbm.at[pl.ds(s * 8, 8), :])

    @pl.when(s == 15)
    def _():
        pltpu.sync_copy(o8_v.at[pl.ds(0, 5), :],
                        out_hbm.at[pl.ds(120, 5), :])


@jax.jit
def _run(x, wtp, bias, src2, dst2):
    mesh = plsc.VectorSubcoreMesh(
        core_axis_name="c", subcore_axis_name="s", num_cores=1,
        num_subcores=16)
    f = pl.kernel(
        _sc_body,
        out_type=jax.ShapeDtypeStruct((125, 128), jnp.float32),
        mesh=mesh,
        scratch_types=[
            pltpu.VMEM((NPS, IN_D), jnp.float32),
            pltpu.VMEM((IN_D // 8, 8 * OUT_D), jnp.float32),
            pltpu.VMEM((OUT_D,), jnp.float32),
            pltpu.VMEM((EG, L), jnp.int32),
            pltpu.VMEM((EG, L), jnp.int32),
            pltpu.VMEM((N_PAD,), jnp.float32),
            pltpu.VMEM((E_PAD, IN_D), jnp.float32),
            pltpu.VMEM((NPS // 8, 8 * OUT_D), jnp.float32),
            pltpu.SemaphoreType.DMA,
        ],
        compiler_params=pltpu.CompilerParams(needs_layout_passes=False),
    )
    return f(x, wtp, bias, src2, dst2)


def kernel(L_args_0_, L_args_1_,
           L_self_modules_encoder_modules_lin_parameters_weight_,
           L_self_modules_encoder_parameters_bias_):
    x = L_args_0_
    edge_index = L_args_1_.astype(jnp.int32)
    weight = L_self_modules_encoder_modules_lin_parameters_weight_
    bias = L_self_modules_encoder_parameters_bias_

    n_edges = edge_index.shape[1]
    src2 = jnp.pad(edge_index[0], (0, E_PAD - n_edges)).reshape(EG, L)
    dst2 = jnp.pad(edge_index[1], (0, E_PAD - n_edges)).reshape(EG, L)
    wtp = weight.T.reshape(IN_D // 8, 8 * OUT_D)

    out8 = _run(x, wtp, bias, src2, dst2)
    return (out8.reshape(N_NODES, OUT_D),)


# speedup vs baseline: 1.2290x; 1.2290x over previous
import jax
import jax.numpy as jnp
from jax import lax
from jax.experimental import pallas as pl
from jax.experimental.pallas import tpu as pltpu
from jax.experimental.pallas import tpu_sc as plsc

N_NODES = 1000
N_PAD = 1024
NPS = 64
LAST = N_NODES - 15 * NPS
E_PAD = 112
EG = E_PAD // 16
IN_D = 128
OUT_D = 16
L = 16


def _rsqrt(q):
    y = plsc.bitcast(
        jnp.int32(0x5F3759DF) - (plsc.bitcast(q, jnp.int32) >> 1),
        jnp.float32)
    for _ in range(3):
        y = y * (1.5 - 0.5 * q * y * y)
    return y


def _sc_body(x_hbm, wtp_hbm, bias_hbm, src_hbm, dst_hbm,
             out_hbm,
             x_v, wtp_v, bias_v, src_v, dst_v,
             degall_v, xrows_v, o8_v, sem_g):
    s = lax.axis_index("s")
    base = s * NPS
    xoff = lax.min(base, jnp.int32(N_NODES - NPS))
    xshift = base - xoff

    pltpu.sync_copy(src_hbm, src_v)
    pltpu.sync_copy(dst_hbm, dst_v)

    gathers = [
        pltpu.async_copy(x_hbm.at[src_v.at[g]],
                         xrows_v.at[pl.ds(g * L, L), :], sem_g)
        for g in range(EG)
    ]

    pltpu.sync_copy(x_hbm.at[pl.ds(xoff, NPS), :], x_v)
    pltpu.sync_copy(wtp_hbm, wtp_v)
    pltpu.sync_copy(bias_hbm, bias_v)

    ones = jnp.full((L,), 1.0, jnp.float32)
    for i in range(N_PAD // L):
        degall_v[pl.ds(i * L, L)] = ones
    for g in range(EG):
        sv = src_v[g, :]
        dv = dst_v[g, :]
        keep = sv != dv
        plsc.addupdate_scatter(degall_v, [dv], ones, mask=keep)

    bias_vec = bias_v[...]

    def mm_body(t, carry):
        n0 = t * 4
        accs = [[jnp.zeros((L,), jnp.float32) for _ in range(4)]
                for _ in range(4)]
        for kb in range(IN_D // L):
            xvs = [x_v[n0 + u + xshift, pl.ds(kb * L, L)]
                   for u in range(4)]
            for j in range(L):
                k = kb * L + j
                wrow = wtp_v[k >> 3, pl.ds((k & 7) * L, L)]
                for u in range(4):
                    b = j % 4
                    accs[u][b] = accs[u][b] + xvs[u][j] * wrow
        for u in range(4):
            n = n0 + u
            a = accs[u]
            o8_v[n >> 3, pl.ds((n & 7) * L, L)] = (a[0] + a[1]) + (a[2] + a[3])
        return carry

    lax.fori_loop(0, NPS // 4, mm_body, 0)

    def scale_body(t, carry):
        n0 = t * L
        dsq = 1.0 / degall_v[pl.ds(base + n0, L)]
        for u in range(L):
            n = n0 + u
            hrow = o8_v[n >> 3, pl.ds((n & 7) * L, L)]
            o8_v[n >> 3, pl.ds((n & 7) * L, L)] = hrow * dsq[u] + bias_vec
        return carry

    lax.fori_loop(0, NPS // L, scale_body, 0)

    for g_desc in gathers:
        g_desc.wait()

    def edge_body(g, carry):
        sv = src_v[g, :]
        dv = dst_v[g, :]
        deg_s = plsc.load_gather(degall_v, [sv])
        deg_d = plsc.load_gather(degall_v, [dv])
        nv = _rsqrt(deg_s * deg_d)
        ldv = dv - base
        for j in range(L):
            lj = ldv[j]
            sj = sv[j]
            dj = dv[j]
            nj = nv[j]

            @pl.when((lj >= 0) & (lj < NPS) & (sj != dj))
            def _():
                erow = g * L + j

                def emm(kb, acc4):
                    xv = xrows_v[erow, pl.ds(kb * L, L)]
                    acc4 = list(acc4)
                    for jj in range(L):
                        k8 = jj
                        wrow = wtp_v[(kb * L + jj) >> 3,
                                     pl.ds(((kb * L + jj) & 7) * L, L)]
                        acc4[jj % 4] = acc4[jj % 4] + xv[jj] * wrow
                    return tuple(acc4)

                a = lax.fori_loop(0, IN_D // L, emm,
                                  tuple(jnp.zeros((L,), jnp.float32)
                                        for _ in range(4)))
                msg = ((a[0] + a[1]) + (a[2] + a[3])) * nj
                cur = o8_v[lj >> 3, pl.ds((lj & 7) * L, L)]
                o8_v[lj >> 3, pl.ds((lj & 7) * L, L)] = cur + msg

        return carry

    lax.fori_loop(0, EG, edge_body, 0)

    @pl.when(s < 15)
    def _():
        pltpu.sync_copy(o8_v, out_hbm.at[pl.ds(s * 8, 8), :])

    @pl.when(s == 15)
    def _():
        pltpu.sync_copy(o8_v.at[pl.ds(0, 5), :],
                        out_hbm.at[pl.ds(120, 5), :])


@jax.jit
def _run(x, wtp, bias, src2, dst2):
    mesh = plsc.VectorSubcoreMesh(
        core_axis_name="c", subcore_axis_name="s", num_cores=1,
        num_subcores=16)
    f = pl.kernel(
        _sc_body,
        out_type=jax.ShapeDtypeStruct((125, 128), jnp.float32),
        mesh=mesh,
        scratch_types=[
            pltpu.VMEM((NPS, IN_D), jnp.float32),
            pltpu.VMEM((IN_D // 8, 8 * OUT_D), jnp.float32),
            pltpu.VMEM((OUT_D,), jnp.float32),
            pltpu.VMEM((EG, L), jnp.int32),
            pltpu.VMEM((EG, L), jnp.int32),
            pltpu.VMEM((N_PAD,), jnp.float32),
            pltpu.VMEM((E_PAD, IN_D), jnp.float32),
            pltpu.VMEM((NPS // 8, 8 * OUT_D), jnp.float32),
            pltpu.SemaphoreType.DMA,
        ],
        compiler_params=pltpu.CompilerParams(needs_layout_passes=False),
    )
    return f(x, wtp, bias, src2, dst2)


def kernel(L_args_0_, L_args_1_,
           L_self_modules_encoder_modules_lin_parameters_weight_,
           L_self_modules_encoder_parameters_bias_):
    x = L_args_0_
    edge_index = L_args_1_.astype(jnp.int32)
    weight = L_self_modules_encoder_modules_lin_parameters_weight_
    bias = L_self_modules_encoder_parameters_bias_

    n_edges = edge_index.shape[1]
    src2 = jnp.pad(edge_index[0], (0, E_PAD - n_edges)).reshape(EG, L)
    dst2 = jnp.pad(edge_index[1], (0, E_PAD - n_edges)).reshape(EG, L)
    wtp = weight.T.reshape(IN_D // 8, 8 * OUT_D)

    out8 = _run(x, wtp, bias, src2, dst2)
    return (out8.reshape(N_NODES, OUT_D),)


# speedup vs baseline: 1.6981x; 1.3817x over previous
import jax
import jax.numpy as jnp
from jax import lax
from jax.experimental import pallas as pl
from jax.experimental.pallas import tpu as pltpu
from jax.experimental.pallas import tpu_sc as plsc

N_NODES = 1000
N_PAD = 1024
NPS = 64
E_PAD = 112
EG = E_PAD // 16
IN_D = 128
OUT_D = 16
L = 16


def _rsqrt(q):
    y = plsc.bitcast(
        jnp.int32(0x5F3759DF) - (plsc.bitcast(q, jnp.int32) >> 1),
        jnp.float32)
    for _ in range(3):
        y = y * (1.5 - 0.5 * q * y * y)
    return y


def _sc_body(x_hbm, wtp_hbm, bias_hbm, src_hbm, dst_hbm,
             out_hbm, h_hbm,
             x_v, wtp_v, bias_v, src_v, dst_v,
             degall_v, h8_v, o8_v, hall_v, sem_e, sem_in, sem_h):
    s = lax.axis_index("s")
    base = s * NPS
    xoff = lax.min(base, jnp.int32(N_NODES - NPS))
    xshift = base - xoff

    cp_src = pltpu.async_copy(src_hbm, src_v, sem_e)
    cp_dst = pltpu.async_copy(dst_hbm, dst_v, sem_e)
    cp_x = pltpu.async_copy(x_hbm.at[pl.ds(xoff, NPS), :], x_v, sem_in)
    cp_wt = pltpu.async_copy(wtp_hbm, wtp_v, sem_in)
    cp_b = pltpu.async_copy(bias_hbm, bias_v, sem_in)

    cp_src.wait()
    cp_dst.wait()

    ones = jnp.full((L,), 1.0, jnp.float32)
    for i in range(N_PAD // L):
        degall_v[pl.ds(i * L, L)] = ones
    for g in range(EG):
        sv = src_v[g, :]
        dv = dst_v[g, :]
        keep = sv != dv
        plsc.addupdate_scatter(degall_v, [dv], ones, mask=keep)

    cp_x.wait()
    cp_wt.wait()
    cp_b.wait()

    def mm_body(t, carry):
        n0 = t * 4
        accs = [[jnp.zeros((L,), jnp.float32) for _ in range(4)]
                for _ in range(4)]
        for kb in range(IN_D // L):
            xvs = [x_v[n0 + u + xshift, pl.ds(kb * L, L)]
                   for u in range(4)]
            for j in range(L):
                k = kb * L + j
                wrow = wtp_v[k >> 3, pl.ds((k & 7) * L, L)]
                for u in range(4):
                    b = j % 4
                    accs[u][b] = accs[u][b] + xvs[u][j] * wrow
        for u in range(4):
            n = n0 + u
            a = accs[u]
            h8_v[n >> 3, pl.ds((n & 7) * L, L)] = (a[0] + a[1]) + (a[2] + a[3])
        return carry

    lax.fori_loop(0, NPS // 4, mm_body, 0)

    pltpu.sync_copy(h8_v, h_hbm.at[pl.ds(s * 8, 8), :])
    plsc.subcore_barrier()
    cp_hall = pltpu.async_copy(h_hbm, hall_v, sem_h)

    bias_vec = bias_v[...]

    def scale_body(t, carry):
        n0 = t * L
        dsq = 1.0 / degall_v[pl.ds(base + n0, L)]
        for u in range(L):
            n = n0 + u
            hrow = h8_v[n >> 3, pl.ds((n & 7) * L, L)]
            o8_v[n >> 3, pl.ds((n & 7) * L, L)] = hrow * dsq[u] + bias_vec
        return carry

    lax.fori_loop(0, NPS // L, scale_body, 0)

    cp_hall.wait()

    def edge_body(g, carry):
        sv = src_v[g, :]
        dv = dst_v[g, :]
        deg_s = plsc.load_gather(degall_v, [sv])
        deg_d = plsc.load_gather(degall_v, [dv])
        nv = _rsqrt(deg_s * deg_d)
        ldv = dv - base
        for j in range(L):
            lj = ldv[j]
            sj = sv[j]
            dj = dv[j]
            nj = nv[j]

            @pl.when((lj >= 0) & (lj < NPS) & (sj != dj))
            def _():
                hrow = hall_v[sj >> 3, pl.ds((sj & 7) * L, L)]
                cur = o8_v[lj >> 3, pl.ds((lj & 7) * L, L)]
                o8_v[lj >> 3, pl.ds((lj & 7) * L, L)] = cur + hrow * nj

        return carry

    lax.fori_loop(0, EG, edge_body, 0)

    @pl.when(s < 15)
    def _():
        pltpu.sync_copy(o8_v, out_hbm.at[pl.ds(s * 8, 8), :])

    @pl.when(s == 15)
    def _():
        pltpu.sync_copy(o8_v.at[pl.ds(0, 5), :],
                        out_hbm.at[pl.ds(120, 5), :])


@jax.jit
def _run(x, wtp, bias, src2, dst2):
    mesh = plsc.VectorSubcoreMesh(
        core_axis_name="c", subcore_axis_name="s", num_cores=1,
        num_subcores=16)
    f = pl.kernel(
        _sc_body,
        out_type=(jax.ShapeDtypeStruct((125, 128), jnp.float32),
                  jax.ShapeDtypeStruct((N_PAD // 8, 8 * OUT_D), jnp.float32)),
        mesh=mesh,
        scratch_types=[
            pltpu.VMEM((NPS, IN_D), jnp.float32),
            pltpu.VMEM((IN_D // 8, 8 * OUT_D), jnp.float32),
            pltpu.VMEM((OUT_D,), jnp.float32),
            pltpu.VMEM((EG, L), jnp.int32),
            pltpu.VMEM((EG, L), jnp.int32),
            pltpu.VMEM((N_PAD,), jnp.float32),
            pltpu.VMEM((NPS // 8, 8 * OUT_D), jnp.float32),
            pltpu.VMEM((NPS // 8, 8 * OUT_D), jnp.float32),
            pltpu.VMEM((N_PAD // 8, 8 * OUT_D), jnp.float32),
            pltpu.SemaphoreType.DMA,
            pltpu.SemaphoreType.DMA,
            pltpu.SemaphoreType.DMA,
        ],
        compiler_params=pltpu.CompilerParams(needs_layout_passes=False),
    )
    return f(x, wtp, bias, src2, dst2)


def kernel(L_args_0_, L_args_1_,
           L_self_modules_encoder_modules_lin_parameters_weight_,
           L_self_modules_encoder_parameters_bias_):
    x = L_args_0_
    edge_index = L_args_1_.astype(jnp.int32)
    weight = L_self_modules_encoder_modules_lin_parameters_weight_
    bias = L_self_modules_encoder_parameters_bias_

    n_edges = edge_index.shape[1]
    src2 = jnp.pad(edge_index[0], (0, E_PAD - n_edges)).reshape(EG, L)
    dst2 = jnp.pad(edge_index[1], (0, E_PAD - n_edges)).reshape(EG, L)
    wtp = weight.T.reshape(IN_D // 8, 8 * OUT_D)

    out8, _ = _run(x, wtp, bias, src2, dst2)
    return (out8.reshape(N_NODES, OUT_D),)
